# baseline (speedup 1.0000x reference)
import functools

import jax
import jax.numpy as jnp
from jax import lax
from jax.experimental import pallas as pl
from jax.experimental.pallas import tpu as pltpu
from jax.experimental.pallas import tpu_sc as plsc

B = 4096
S = 200
EMBED = 64
CHUNK = 100
NUM_CHUNKS = B * 2
POS_ROWS = 200

_info = plsc.get_sparse_core_info()
NC, NS = _info.num_cores, _info.num_subcores
NW = NC * NS
CHUNKS_PER_W = NUM_CHUNKS // NW


def _body(table_hbm, x_hbm, pos_hbm, out_hbm,
          pos_v, idx_v, rows_v, st_v, gsem0, gsem1, ssem0, ssem1):
    gsem = (gsem0, gsem1)
    ssem = (ssem0, ssem1)
    wid = lax.axis_index("s") * NC + lax.axis_index("c")
    c_base = wid * CHUNKS_PER_W
    pltpu.sync_copy(pos_hbm.at[pl.ds(0, POS_ROWS)], pos_v)
    pltpu.sync_copy(x_hbm.at[pl.ds(c_base, CHUNKS_PER_W)], idx_v)

    def start_gather(t, slot):
        tw = lax.rem(t, CHUNKS_PER_W)
        pltpu.async_copy(
            table_hbm.at[idx_v.at[tw]], rows_v.at[slot], gsem[slot])

    def wait_gather(slot):
        pltpu.make_async_copy(
            table_hbm.at[idx_v.at[0]], rows_v.at[slot], gsem[slot]).wait()

    def out_slice(t):
        c = c_base + t
        return out_hbm.at[c // 2, pl.ds(lax.rem(c, 2) * CHUNK, CHUNK),
                          pl.ds(0, EMBED)]

    def start_store(t, slot):
        pltpu.async_copy(st_v.at[slot], out_slice(t), ssem[slot])

    def wait_store(slot):
        pltpu.make_async_copy(st_v.at[slot], out_slice(0), ssem[slot]).wait()

    start_gather(0, 0)

    def pair_body(i, carry):
        for b in (0, 1):
            t = i * 2 + b
            start_gather(t + 1, 1 - b)
            wait_gather(b)
            s0 = lax.rem(c_base + t, 2) * CHUNK

            @pl.when(i > 0)
            def _():
                wait_store(b)

            @plsc.parallel_loop(0, CHUNK, unroll=8)
            def _(r):
                for j in range(EMBED // 16):
                    sl = pl.ds(j * 16, 16)
                    st_v[b, r, sl] = rows_v[b, r, sl] + pos_v[s0 + r, sl]

            start_store(t, b)
        return carry

    lax.fori_loop(0, CHUNKS_PER_W // 2, pair_body, 0)
    wait_store(0)
    wait_store(1)
    wait_gather(0)


@jax.jit
def _run(x2, token_table, pos_table):
    mesh = plsc.VectorSubcoreMesh(core_axis_name="c", subcore_axis_name="s")
    k = functools.partial(
        pl.kernel,
        out_type=jax.ShapeDtypeStruct((B, S, 128), jnp.float32),
        mesh=mesh,
        scratch_types=[
            pltpu.VMEM((POS_ROWS, EMBED), jnp.float32),
            pltpu.VMEM((CHUNKS_PER_W, CHUNK), jnp.int32),
            pltpu.VMEM((2, CHUNK, EMBED), jnp.float32),
            pltpu.VMEM((2, CHUNK, EMBED), jnp.float32),
            pltpu.SemaphoreType.DMA,
            pltpu.SemaphoreType.DMA,
            pltpu.SemaphoreType.DMA,
            pltpu.SemaphoreType.DMA,
        ],
        compiler_params=pltpu.CompilerParams(use_tc_tiling_on_sc=False),
    )(_body)
    out_pad = k(token_table, x2, pos_table)
    return out_pad[:, :, :EMBED]


def kernel(x, token_table, pos_table):
    x2 = x.reshape(NUM_CHUNKS, CHUNK).astype(jnp.int32)
    return _run(x2, token_table, pos_table)


# speedup vs baseline: 1.1552x; 1.1552x over previous
import functools

import jax
import jax.numpy as jnp
from jax import lax
from jax.experimental import pallas as pl
from jax.experimental.pallas import tpu as pltpu
from jax.experimental.pallas import tpu_sc as plsc

B = 4096
S = 200
EMBED = 64
CHUNK = 128
TOTAL = B * S
NUM_CHUNKS = TOTAL // CHUNK
POS_ROWS = 320
RING = 4

_info = plsc.get_sparse_core_info()
NC, NS = _info.num_cores, _info.num_subcores
NW = NC * NS
CHUNKS_PER_W = NUM_CHUNKS // NW


def _body(table_hbm, x_hbm, pos_hbm, out_hbm, pos_v, idx_v, rows_v, st_v,
          *sems):
    gsem = sems[:RING]
    ssem = sems[RING:]
    wid = lax.axis_index("s") * NC + lax.axis_index("c")
    c_base = wid * CHUNKS_PER_W
    pltpu.sync_copy(pos_hbm, pos_v)
    pltpu.sync_copy(x_hbm.at[pl.ds(c_base, CHUNKS_PER_W)], idx_v)

    def start_gather(t, slot):
        tw = lax.rem(t, CHUNKS_PER_W)
        pltpu.async_copy(
            table_hbm.at[idx_v.at[tw]], rows_v.at[slot], gsem[slot])

    def wait_gather(slot):
        pltpu.make_async_copy(
            table_hbm.at[idx_v.at[0]], rows_v.at[slot], gsem[slot]).wait()

    def start_store(t, slot):
        pltpu.async_copy(
            st_v.at[slot],
            out_hbm.at[pl.ds((c_base + t) * CHUNK, CHUNK), pl.ds(0, EMBED)],
            ssem[slot])

    def wait_store(slot):
        pltpu.make_async_copy(
            st_v.at[slot],
            out_hbm.at[pl.ds(0, CHUNK), pl.ds(0, EMBED)], ssem[slot]).wait()

    for p in range(RING - 1):
        start_gather(p, p)

    def ring_body(i, carry):
        for b in range(RING):
            t = i * RING + b
            start_gather(t + RING - 1, (b + RING - 1) % RING)
            wait_gather(b)
            p0 = lax.rem((c_base + t) * CHUNK, S)

            @pl.when(i > 0)
            def _():
                wait_store(b)

            @plsc.parallel_loop(0, CHUNK, unroll=8)
            def _(r):
                for j in range(EMBED // 16):
                    sl = pl.ds(j * 16, 16)
                    st_v[b, r, sl] = rows_v[b, r, sl] + pos_v[p0 + r, sl]

            start_store(t, b)
        return carry

    lax.fori_loop(0, CHUNKS_PER_W // RING, ring_body, 0)
    for b in range(RING):
        wait_store(b)
    for p in range(RING - 1):
        wait_gather((CHUNKS_PER_W + p) % RING)


@jax.jit
def _run(x2, token_table, pos_ext):
    mesh = plsc.VectorSubcoreMesh(core_axis_name="c", subcore_axis_name="s")
    k = functools.partial(
        pl.kernel,
        out_type=jax.ShapeDtypeStruct((TOTAL, 128), jnp.float32),
        mesh=mesh,
        scratch_types=[
            pltpu.VMEM((POS_ROWS, EMBED), jnp.float32),
            pltpu.VMEM((CHUNKS_PER_W, CHUNK), jnp.int32),
            pltpu.VMEM((RING, CHUNK, EMBED), jnp.float32),
            pltpu.VMEM((RING, CHUNK, EMBED), jnp.float32),
        ] + [pltpu.SemaphoreType.DMA] * (2 * RING),
        compiler_params=pltpu.CompilerParams(use_tc_tiling_on_sc=False),
    )(_body)
    out_pad = k(token_table, x2, pos_ext)
    return out_pad.reshape(B, S, 128)[:, :, :EMBED]


def kernel(x, token_table, pos_table):
    x2 = x.reshape(TOTAL // 128, 128).astype(jnp.int32)
    pos_ext = jnp.concatenate([pos_table[:S], pos_table[: POS_ROWS - S]], axis=0)
    return _run(x2, token_table, pos_ext)
